# Initial kernel scaffold; baseline (speedup 1.0000x reference)
import jax
import jax.numpy as jnp
from jax.experimental import pallas as pl


def kernel(text_x, text_edge_index, text_batch, scene_x, scene_edge_index, scene_batch, W_text, b_text, W_scene, b_scene):
    raise NotImplementedError("write your pallas kernel here")


# speedup vs baseline: 12.5263x; 12.5263x over previous
import jax
import jax.numpy as jnp
from jax import lax
from jax.experimental import pallas as pl
from jax.experimental.pallas import tpu as pltpu
from jax.experimental.pallas import tpu_sc as plsc

N = 10000
E = 320000
D = 128
B = 100

DH = D // 2
NPAD = 10240
RT = NPAD // 16
RC = 128
EC = 128
EPAD = 321536
ETP = EPAD // 16
NCH_E = ETP // EC
NCH_R = RT // RC


def _fast_rsqrt(d):
    xi = lax.bitcast_convert_type(d, jnp.int32)
    xi = jnp.int32(0x5F3759DF) - (xi >> 1)
    r = lax.bitcast_convert_type(xi, jnp.float32)
    r = r * (1.5 - 0.5 * d * r * r)
    r = r * (1.5 - 0.5 * d * r * r)
    r = r * (1.5 - 0.5 * d * r * r)
    return r


def _sc_gcn_body(xw0, xw1, ei, bs, out0, out1, y0, y1, acc, degsp,
                 sidx, didx, rows, ones, degl, dinv, rbuf, bvec):
    c = lax.axis_index("c")
    s = lax.axis_index("s")

    def _zero_body(r, _):
        degl[r, :] = jnp.zeros((16,), jnp.float32)
        return _
    lax.fori_loop(0, RT, _zero_body, None)

    def _ones_body(r, _):
        ones[r, :] = jnp.ones((16,), jnp.float32)
        return _
    lax.fori_loop(0, EC, _ones_body, None)

    pltpu.sync_copy(degl, degsp.at[pl.ds(RT * s, RT)])
    plsc.subcore_barrier()

    ebase = s * ETP

    def _deg_body(j, _):
        off = pl.multiple_of(ebase + j * EC, 8)
        pltpu.sync_copy(ei.at[c, 1, pl.ds(off, EC)], didx.at[0])
        pltpu.sync_copy(ones, degsp.at[didx.at[0]], add=True)
        return _
    lax.fori_loop(0, NCH_E, _deg_body, None)
    plsc.subcore_barrier()

    pltpu.sync_copy(degsp.at[pl.ds(RT * s, RT)], degl)

    def _rs_body(r, _):
        d = degl[r, :] + 1.0
        dinv[r, :] = _fast_rsqrt(d)
        return _
    lax.fori_loop(0, RT, _rs_body, None)

    rbase = s * RT
    for p, xwp, yp, outp in ((0, xw0, y0, out0), (1, xw1, y1, out1)):
        pltpu.sync_copy(bs.at[c, pl.ds(p * DH, DH)], bvec)

        for rj in range(NCH_R):
            row0 = rbase + rj * RC
            pltpu.sync_copy(xwp.at[c, pl.ds(row0, RC)], rbuf)

            def _scale_body(r, _, rj=rj):
                sv = dinv[rj * RC + r, :]
                for k in range(DH // 16):
                    rbuf[r, pl.ds(k * 16, 16)] = rbuf[r, pl.ds(k * 16, 16)] * sv
                return _
            lax.fori_loop(0, RC, _scale_body, None)
            pltpu.sync_copy(rbuf, yp.at[pl.ds(c * NPAD + row0, RC)])
            pltpu.sync_copy(rbuf, acc.at[pl.ds(row0, RC)])
        plsc.subcore_barrier()

        def _edge_body(j, _):
            off = pl.multiple_of(ebase + j * EC, 8)
            pltpu.sync_copy(ei.at[c, 0, pl.ds(off, EC)], sidx)
            pltpu.sync_copy(ei.at[c, 1, pl.ds(off, EC)], didx.at[0])
            boff = c * NPAD
            for i in range(EC // 16):
                sidx[pl.ds(i * 16, 16)] = sidx[pl.ds(i * 16, 16)] + boff
            pltpu.sync_copy(yp.at[sidx], rows)
            pltpu.sync_copy(rows, acc.at[didx.at[0]], add=True)
            return _
        lax.fori_loop(0, NCH_E, _edge_body, None)
        plsc.subcore_barrier()

        for rj in range(NCH_R):
            row0 = rbase + rj * RC
            pltpu.sync_copy(acc.at[pl.ds(row0, RC)], rbuf)

            def _out_body(r, _, rj=rj):
                sv = dinv[rj * RC + r, :]
                for k in range(DH // 16):
                    rbuf[r, pl.ds(k * 16, 16)] = (
                        rbuf[r, pl.ds(k * 16, 16)] * sv + bvec[pl.ds(k * 16, 16)])
                return _
            lax.fori_loop(0, RC, _out_body, None)
            pltpu.sync_copy(rbuf, outp.at[c, pl.ds(row0, RC)])
        plsc.subcore_barrier()


@jax.jit
def _sc_gcn(xw0, xw1, ei, bs):
    mesh = plsc.VectorSubcoreMesh(core_axis_name="c", subcore_axis_name="s")
    f = pl.kernel(
        _sc_gcn_body,
        out_type=[jax.ShapeDtypeStruct((2, NPAD, DH), jnp.float32),
                  jax.ShapeDtypeStruct((2, NPAD, DH), jnp.float32),
                  jax.ShapeDtypeStruct((2 * NPAD, DH), jnp.float32),
                  jax.ShapeDtypeStruct((2 * NPAD, DH), jnp.float32)],
        mesh=mesh,
        compiler_params=pltpu.CompilerParams(use_tc_tiling_on_sc=False),
        scratch_types=[
            pltpu.VMEM_SHARED((NPAD, DH), jnp.float32),
            pltpu.VMEM_SHARED((NPAD, 16), jnp.float32),
            pltpu.VMEM((EC,), jnp.int32),
            pltpu.VMEM((1, EC), jnp.int32),
            pltpu.VMEM((EC, DH), jnp.float32),
            pltpu.VMEM((EC, 16), jnp.float32),
            pltpu.VMEM((RT, 16), jnp.float32),
            pltpu.VMEM((RT, 16), jnp.float32),
            pltpu.VMEM((RC, DH), jnp.float32),
            pltpu.VMEM((DH,), jnp.float32),
        ],
    )
    return f(xw0, xw1, ei, bs)


def _mm_body(x_ref, w_ref, o0_ref, o1_ref):
    r = jnp.dot(x_ref[0], w_ref[0], preferred_element_type=jnp.float32)
    o0_ref[0] = r[:, :DH]
    o1_ref[0] = r[:, DH:]


@jax.jit
def _tc_matmul(xs, Ws):
    BM = 1024
    return pl.pallas_call(
        _mm_body,
        grid=(2, NPAD // BM),
        in_specs=[pl.BlockSpec((1, BM, D), lambda b, i: (b, i, 0)),
                  pl.BlockSpec((1, D, D), lambda b, i: (b, 0, 0))],
        out_specs=[pl.BlockSpec((1, BM, DH), lambda b, i: (b, i, 0)),
                   pl.BlockSpec((1, BM, DH), lambda b, i: (b, i, 0))],
        out_shape=[jax.ShapeDtypeStruct((2, NPAD, DH), jnp.float32),
                   jax.ShapeDtypeStruct((2, NPAD, DH), jnp.float32)],
    )(xs, Ws)


def _pool_body(x_ref, o_ref):
    o_ref[...] = jnp.mean(x_ref[...], axis=1)


@jax.jit
def _tc_pool(cw):
    return pl.pallas_call(
        _pool_body,
        out_shape=jax.ShapeDtypeStruct((B, D), jnp.float32),
    )(cw)


def kernel(text_x, text_edge_index, text_batch, scene_x, scene_edge_index,
           scene_batch, W_text, b_text, W_scene, b_scene):
    xs = jnp.zeros((2, NPAD, D), jnp.float32).at[:, :N, :].set(
        jnp.stack([text_x, scene_x]))
    Ws = jnp.stack([W_text, W_scene])
    bs = jnp.stack([b_text, b_scene])
    ei = jnp.stack([text_edge_index, scene_edge_index])
    ei_p = jnp.concatenate(
        [ei, jnp.full((2, 2, EPAD - E), NPAD - 1, ei.dtype)], axis=-1)

    xw0, xw1 = _tc_matmul(xs, Ws)
    out0, out1, _, _ = _sc_gcn(xw0, xw1, ei_p, bs)
    out = jnp.concatenate([out0, out1], axis=-1)

    contextual_words = out[0, :N].reshape(B, N // B, D)
    scene_graph_feats = out[1, :N].reshape(B, N // B, D)
    question = _tc_pool(contextual_words)
    return contextual_words, question, scene_graph_feats


# speedup vs baseline: 20.9387x; 1.6716x over previous
import jax
import jax.numpy as jnp
from jax import lax
from jax.experimental import pallas as pl
from jax.experimental.pallas import tpu as pltpu
from jax.experimental.pallas import tpu_sc as plsc

N = 10000
E = 320000
D = 128
B = 100

DH = D // 2
NPAD = 10240
RT = NPAD // 16
RC = 128
EC = 128
EPAD = 321536
ETP = EPAD // 16
NCH_E = ETP // EC
NCH_R = RT // RC


def _fast_rsqrt(d):
    xi = lax.bitcast_convert_type(d, jnp.int32)
    xi = jnp.int32(0x5F3759DF) - (xi >> 1)
    r = lax.bitcast_convert_type(xi, jnp.float32)
    r = r * (1.5 - 0.5 * d * r * r)
    r = r * (1.5 - 0.5 * d * r * r)
    r = r * (1.5 - 0.5 * d * r * r)
    return r


def _sc_gcn_body(xw0, xw1, eir, bs, out0, out1, y0, y1, acc, degsp,
                 sidx, didx, rowsA, rowsB, ones, dinv, rbufA, rbufB, bvec,
                 semG, semH, semS, semT, semD, semW, semI):
    c = lax.axis_index("c")
    s = lax.axis_index("s")
    boff = c * NPAD

    def _zero_body(r, _):
        dinv[r, :] = jnp.zeros((16,), jnp.float32)
        return _
    lax.fori_loop(0, RT, _zero_body, None)

    def _ones_body(r, _):
        ones[r, :] = jnp.ones((16,), jnp.float32)
        return _
    lax.fori_loop(0, EC, _ones_body, None)

    pltpu.sync_copy(dinv, degsp.at[pl.ds(RT * s, RT)])
    plsc.subcore_barrier()

    def _didx_load(j, sem):
        pltpu.async_copy(eir.at[c, 1, s, j], didx.at[j % 4], sem)

    def _didx_wait(j, sem):
        pltpu.make_async_copy(eir.at[c, 1, s, j], didx.at[j % 4], sem).wait()

    _didx_load(0, semI)

    def _deg_body(j, _):
        _didx_wait(j, semI)
        pltpu.async_copy(ones, degsp.at[didx.at[j % 4]], semD, add=True)

        @pl.when(j >= 1)
        def _():
            pltpu.make_async_copy(ones, degsp.at[didx.at[j % 4]], semD).wait()

        @pl.when(j + 1 < NCH_E)
        def _():
            _didx_load(j + 1, semI)
        return _
    lax.fori_loop(0, NCH_E, _deg_body, None)
    pltpu.make_async_copy(ones, degsp.at[didx.at[0]], semD).wait()
    plsc.subcore_barrier()

    pltpu.sync_copy(degsp.at[pl.ds(RT * s, RT)], dinv)

    def _rs_body(r, _):
        d = dinv[r, :] + 1.0
        dinv[r, :] = _fast_rsqrt(d)
        return _
    lax.fori_loop(0, RT, _rs_body, None)

    rbase = s * RT
    bufs = (rbufA, rbufB)
    for p, xwp, yp, outp in ((0, xw0, y0, out0), (1, xw1, y1, out1)):
        pltpu.sync_copy(bs.at[c, pl.ds(p * DH, DH)], bvec)

        pltpu.async_copy(xwp.at[c, pl.ds(rbase, RC)], rbufA, semG)
        for rj in range(NCH_R):
            rb = bufs[rj % 2]
            row0 = rbase + rj * RC
            pltpu.make_async_copy(xwp.at[c, pl.ds(row0, RC)], rb, semG).wait()
            if rj + 1 < NCH_R:
                pltpu.async_copy(
                    xwp.at[c, pl.ds(row0 + RC, RC)], bufs[(rj + 1) % 2], semG)

            def _scale_body(r, _, rj=rj, rb=rb):
                sv = dinv[rj * RC + r, :]
                for k in range(DH // 16):
                    rb[r, pl.ds(k * 16, 16)] = rb[r, pl.ds(k * 16, 16)] * sv
                return _
            lax.fori_loop(0, RC, _scale_body, None)
            pltpu.async_copy(rb, yp.at[pl.ds(c * NPAD + row0, RC)], semW)
            pltpu.sync_copy(rb, acc.at[pl.ds(row0, RC)])
            pltpu.make_async_copy(rb, yp.at[pl.ds(c * NPAD + row0, RC)], semW).wait()
        plsc.subcore_barrier()

        def _sidx_load(j, sem, yp=yp):
            pltpu.async_copy(eir.at[c, 0, s, j], sidx.at[j % 4], sem)

        def _idx_wait(j, sem, yp=yp):
            pltpu.make_async_copy(eir.at[c, 0, s, j], sidx.at[j % 4], sem).wait()
            pltpu.make_async_copy(eir.at[c, 1, s, j], didx.at[j % 4], sem).wait()

        def _gather(j, rows, sem, yp=yp):
            for k in range(EC // 16):
                sidx[j % 4, pl.ds(k * 16, 16)] = (
                    sidx[j % 4, pl.ds(k * 16, 16)] + boff)
            pltpu.async_copy(yp.at[sidx.at[j % 4]], rows, sem)

        def _gather_wait(j, rows, sem, yp=yp):
            pltpu.make_async_copy(yp.at[sidx.at[j % 4]], rows, sem).wait()

        def _scat(j, rows, sem):
            pltpu.async_copy(rows, acc.at[didx.at[j % 4]], sem, add=True)

        def _scat_wait(j, rows, sem):
            pltpu.make_async_copy(rows, acc.at[didx.at[j % 4]], sem).wait()

        for j in (0, 1):
            _sidx_load(j, semI)
            _didx_load(j, semI)
        for j in (0, 1):
            _idx_wait(j, semI)
        _gather(0, rowsA, semG)
        _gather(1, rowsB, semH)
        for j in (2, 3):
            _sidx_load(j, semI)
            _didx_load(j, semI)

        def _edge_body(i, _, yp=yp):
            j0 = 2 * i
            _gather_wait(j0, rowsA, semG)
            _scat(j0, rowsA, semS)
            _gather_wait(j0 + 1, rowsB, semH)
            _scat(j0 + 1, rowsB, semT)

            @pl.when(j0 + 2 < NCH_E)
            def _():
                _idx_wait(j0 + 2, semI)

            @pl.when(j0 + 3 < NCH_E)
            def _():
                _idx_wait(j0 + 3, semI)
            _scat_wait(j0, rowsA, semS)

            @pl.when(j0 + 2 < NCH_E)
            def _():
                _gather(j0 + 2, rowsA, semG)
            _scat_wait(j0 + 1, rowsB, semT)

            @pl.when(j0 + 3 < NCH_E)
            def _():
                _gather(j0 + 3, rowsB, semH)

            @pl.when(j0 + 4 < NCH_E)
            def _():
                _sidx_load(j0 + 4, semI)
                _didx_load(j0 + 4, semI)

            @pl.when(j0 + 5 < NCH_E)
            def _():
                _sidx_load(j0 + 5, semI)
                _didx_load(j0 + 5, semI)
            return _
        lax.fori_loop(0, NCH_E // 2, _edge_body, None)
        jl = NCH_E - 1
        _gather_wait(jl, rowsA, semG)
        pltpu.sync_copy(rowsA, acc.at[didx.at[jl % 4]], add=True)
        plsc.subcore_barrier()

        pltpu.async_copy(acc.at[pl.ds(rbase, RC)], rbufA, semG)
        for rj in range(NCH_R):
            rb = bufs[rj % 2]
            row0 = rbase + rj * RC
            pltpu.make_async_copy(acc.at[pl.ds(row0, RC)], rb, semG).wait()
            if rj + 1 < NCH_R:
                pltpu.async_copy(
                    acc.at[pl.ds(row0 + RC, RC)], bufs[(rj + 1) % 2], semG)

            def _out_body(r, _, rj=rj, rb=rb):
                sv = dinv[rj * RC + r, :]
                for k in range(DH // 16):
                    rb[r, pl.ds(k * 16, 16)] = (
                        rb[r, pl.ds(k * 16, 16)] * sv + bvec[pl.ds(k * 16, 16)])
                return _
            lax.fori_loop(0, RC, _out_body, None)
            pltpu.async_copy(rb, outp.at[c, pl.ds(row0, RC)], semW)
            pltpu.make_async_copy(rb, outp.at[c, pl.ds(row0, RC)], semW).wait()
        plsc.subcore_barrier()


@jax.jit
def _sc_gcn(xw0, xw1, eir, bs):
    mesh = plsc.VectorSubcoreMesh(core_axis_name="c", subcore_axis_name="s")
    f = pl.kernel(
        _sc_gcn_body,
        out_type=[jax.ShapeDtypeStruct((2, NPAD, DH), jnp.float32),
                  jax.ShapeDtypeStruct((2, NPAD, DH), jnp.float32),
                  jax.ShapeDtypeStruct((2 * NPAD, DH), jnp.float32),
                  jax.ShapeDtypeStruct((2 * NPAD, DH), jnp.float32)],
        mesh=mesh,
        compiler_params=pltpu.CompilerParams(use_tc_tiling_on_sc=False),
        scratch_types=[
            pltpu.VMEM_SHARED((NPAD, DH), jnp.float32),
            pltpu.VMEM_SHARED((NPAD, 16), jnp.float32),
            pltpu.VMEM((4, EC), jnp.int32),
            pltpu.VMEM((4, EC), jnp.int32),
            pltpu.VMEM((EC, DH), jnp.float32),
            pltpu.VMEM((EC, DH), jnp.float32),
            pltpu.VMEM((EC, 16), jnp.float32),
            pltpu.VMEM((RT, 16), jnp.float32),
            pltpu.VMEM((RC, DH), jnp.float32),
            pltpu.VMEM((RC, DH), jnp.float32),
            pltpu.VMEM((DH,), jnp.float32),
            pltpu.SemaphoreType.DMA,
            pltpu.SemaphoreType.DMA,
            pltpu.SemaphoreType.DMA,
            pltpu.SemaphoreType.DMA,
            pltpu.SemaphoreType.DMA,
            pltpu.SemaphoreType.DMA,
            pltpu.SemaphoreType.DMA,
        ],
    )
    return f(xw0, xw1, eir, bs)


def _mm_body(x_ref, w_ref, o0_ref, o1_ref):
    r = jnp.dot(x_ref[0], w_ref[0], preferred_element_type=jnp.float32)
    o0_ref[0] = r[:, :DH]
    o1_ref[0] = r[:, DH:]


@jax.jit
def _tc_matmul(xs, Ws):
    BM = 1024
    return pl.pallas_call(
        _mm_body,
        grid=(2, NPAD // BM),
        in_specs=[pl.BlockSpec((1, BM, D), lambda b, i: (b, i, 0)),
                  pl.BlockSpec((1, D, D), lambda b, i: (b, 0, 0))],
        out_specs=[pl.BlockSpec((1, BM, DH), lambda b, i: (b, i, 0)),
                   pl.BlockSpec((1, BM, DH), lambda b, i: (b, i, 0))],
        out_shape=[jax.ShapeDtypeStruct((2, NPAD, DH), jnp.float32),
                   jax.ShapeDtypeStruct((2, NPAD, DH), jnp.float32)],
    )(xs, Ws)


def _pool_body(x_ref, o_ref):
    o_ref[...] = jnp.mean(x_ref[...], axis=1)


@jax.jit
def _tc_pool(cw):
    return pl.pallas_call(
        _pool_body,
        out_shape=jax.ShapeDtypeStruct((B, D), jnp.float32),
    )(cw)


def kernel(text_x, text_edge_index, text_batch, scene_x, scene_edge_index,
           scene_batch, W_text, b_text, W_scene, b_scene):
    xs = jnp.zeros((2, NPAD, D), jnp.float32).at[:, :N, :].set(
        jnp.stack([text_x, scene_x]))
    Ws = jnp.stack([W_text, W_scene])
    bs = jnp.stack([b_text, b_scene])
    ei = jnp.stack([text_edge_index, scene_edge_index])
    ei_p = jnp.concatenate(
        [ei, jnp.full((2, 2, EPAD - E), NPAD - 1, ei.dtype)], axis=-1)
    eir = ei_p.reshape(2, 2, 16, NCH_E, EC)

    xw0, xw1 = _tc_matmul(xs, Ws)
    out0, out1, _, _ = _sc_gcn(xw0, xw1, eir, bs)
    out = jnp.concatenate([out0, out1], axis=-1)

    contextual_words = out[0, :N].reshape(B, N // B, D)
    scene_graph_feats = out[1, :N].reshape(B, N // B, D)
    question = _tc_pool(contextual_words)
    return contextual_words, question, scene_graph_feats


# speedup vs baseline: 27.4597x; 1.3114x over previous
import jax
import jax.numpy as jnp
from jax import lax
from jax.experimental import pallas as pl
from jax.experimental.pallas import tpu as pltpu
from jax.experimental.pallas import tpu_sc as plsc

N = 10000
E = 320000
D = 128
B = 100

DH = D // 2
NPAD = 10240
RT = NPAD // 16
RC = 128
EC = 128
EPAD = 321536
ETP = EPAD // 16
NCH_E = ETP // EC
NCH_R = RT // RC


def _fast_rsqrt(d):
    xi = lax.bitcast_convert_type(d, jnp.int32)
    xi = jnp.int32(0x5F3759DF) - (xi >> 1)
    r = lax.bitcast_convert_type(xi, jnp.float32)
    r = r * (1.5 - 0.5 * d * r * r)
    r = r * (1.5 - 0.5 * d * r * r)
    r = r * (1.5 - 0.5 * d * r * r)
    return r


def _sc_gcn_body(xw0, xw1, eir, bs, out0, out1, y0, y1, acc, degsp,
                 sidx, didx, rows0, rows1, rows2, rows3, ones, dinv,
                 rbufA, rbufB, bvec,
                 semG0, semG1, semG2, semG3, semS0, semS1, semS2, semS3,
                 semD, semW, semI):
    c = lax.axis_index("c")
    s = lax.axis_index("s")
    boff = c * NPAD
    rows = (rows0, rows1, rows2, rows3)
    semG = (semG0, semG1, semG2, semG3)
    semS = (semS0, semS1, semS2, semS3)

    def _zero_body(r, _):
        dinv[r, :] = jnp.zeros((16,), jnp.float32)
        return _
    lax.fori_loop(0, RT, _zero_body, None)

    def _ones_body(r, _):
        ones[r, :] = jnp.ones((16,), jnp.float32)
        return _
    lax.fori_loop(0, EC, _ones_body, None)

    pltpu.sync_copy(dinv, degsp.at[pl.ds(RT * s, RT)])
    plsc.subcore_barrier()

    def _didx_load(j, sem):
        pltpu.async_copy(eir.at[c, 1, s, j], didx.at[j % 8], sem)

    def _didx_wait(j, sem):
        pltpu.make_async_copy(eir.at[c, 1, s, j], didx.at[j % 8], sem).wait()

    for j in range(4):
        _didx_load(j, semI)
    for j in range(4):
        _didx_wait(j, semI)
    for j in range(4, 8):
        _didx_load(j, semI)

    def _deg_body(q, _):
        j0 = 4 * q
        for b in range(4):
            pltpu.async_copy(ones, degsp.at[didx.at[(j0 + b) % 8]], semD,
                             add=True)
        for b in range(4):
            @pl.when(j0 + 4 + b < NCH_E)
            def _(b=b):
                _didx_wait(j0 + 4 + b, semI)
        for b in range(4):
            pltpu.make_async_copy(ones, degsp.at[didx.at[(j0 + b) % 8]],
                                  semD).wait()
        for b in range(4):
            @pl.when(j0 + 8 + b < NCH_E)
            def _(b=b):
                _didx_load(j0 + 8 + b, semI)
        return _
    lax.fori_loop(0, (NCH_E - 1) // 4, _deg_body, None)
    jl = NCH_E - 1
    pltpu.sync_copy(ones, degsp.at[didx.at[jl % 8]], add=True)
    plsc.subcore_barrier()

    pltpu.sync_copy(degsp.at[pl.ds(RT * s, RT)], dinv)

    def _rs_body(r, _):
        d = dinv[r, :] + 1.0
        dinv[r, :] = _fast_rsqrt(d)
        return _
    lax.fori_loop(0, RT, _rs_body, None)

    rbase = s * RT
    bufs = (rbufA, rbufB)
    for p, xwp, yp, outp in ((0, xw0, y0, out0), (1, xw1, y1, out1)):
        pltpu.sync_copy(bs.at[c, pl.ds(p * DH, DH)], bvec)

        pltpu.async_copy(xwp.at[c, pl.ds(rbase, RC)], rbufA, semG0)
        for rj in range(NCH_R):
            rb = bufs[rj % 2]
            row0 = rbase + rj * RC
            pltpu.make_async_copy(xwp.at[c, pl.ds(row0, RC)], rb, semG0).wait()
            if rj + 1 < NCH_R:
                pltpu.async_copy(
                    xwp.at[c, pl.ds(row0 + RC, RC)], bufs[(rj + 1) % 2], semG0)

            def _scale_body(r, _, rj=rj, rb=rb):
                sv = dinv[rj * RC + r, :]
                for k in range(DH // 16):
                    rb[r, pl.ds(k * 16, 16)] = rb[r, pl.ds(k * 16, 16)] * sv
                return _
            lax.fori_loop(0, RC, _scale_body, None)
            pltpu.async_copy(rb, yp.at[pl.ds(c * NPAD + row0, RC)], semW)
            pltpu.sync_copy(rb, acc.at[pl.ds(row0, RC)])
            pltpu.make_async_copy(rb, yp.at[pl.ds(c * NPAD + row0, RC)], semW).wait()
        plsc.subcore_barrier()

        def _sidx_load(j, sem, yp=yp):
            pltpu.async_copy(eir.at[c, 0, s, j], sidx.at[j % 8], sem)

        def _idx_wait(j, sem, yp=yp):
            pltpu.make_async_copy(eir.at[c, 0, s, j], sidx.at[j % 8], sem).wait()
            pltpu.make_async_copy(eir.at[c, 1, s, j], didx.at[j % 8], sem).wait()

        def _gather(j, rb, sem, yp=yp):
            for k in range(EC // 16):
                sidx[j % 8, pl.ds(k * 16, 16)] = (
                    sidx[j % 8, pl.ds(k * 16, 16)] + boff)
            pltpu.async_copy(yp.at[sidx.at[j % 8]], rb, sem)

        def _gather_wait(j, rb, sem, yp=yp):
            pltpu.make_async_copy(yp.at[sidx.at[j % 8]], rb, sem).wait()

        def _scat(j, rb, sem):
            pltpu.async_copy(rb, acc.at[didx.at[j % 8]], sem, add=True)

        def _scat_wait(j, rb, sem):
            pltpu.make_async_copy(rb, acc.at[didx.at[j % 8]], sem).wait()

        for j in range(4):
            _sidx_load(j, semI)
            _didx_load(j, semI)
        for j in range(4):
            _idx_wait(j, semI)
        for b in range(4):
            _gather(b, rows[b], semG[b])
        for j in range(4, 8):
            _sidx_load(j, semI)
            _didx_load(j, semI)

        def _edge_body(q, _, yp=yp):
            j0 = 4 * q
            for b in range(4):
                _gather_wait(j0 + b, rows[b], semG[b])
                _scat(j0 + b, rows[b], semS[b])
            for b in range(4):
                @pl.when(j0 + 4 + b < NCH_E)
                def _(b=b):
                    _idx_wait(j0 + 4 + b, semI)
            for b in range(4):
                _scat_wait(j0 + b, rows[b], semS[b])

                @pl.when(j0 + 4 + b < NCH_E)
                def _(b=b):
                    _gather(j0 + 4 + b, rows[b], semG[b])
            for b in range(4):
                @pl.when(j0 + 8 + b < NCH_E)
                def _(b=b):
                    _sidx_load(j0 + 8 + b, semI)
                    _didx_load(j0 + 8 + b, semI)
            return _
        lax.fori_loop(0, (NCH_E - 1) // 4, _edge_body, None)
        jl = NCH_E - 1
        _gather_wait(jl, rows[0], semG[0])
        pltpu.sync_copy(rows[0], acc.at[didx.at[jl % 8]], add=True)
        plsc.subcore_barrier()

        pltpu.async_copy(acc.at[pl.ds(rbase, RC)], rbufA, semG0)
        for rj in range(NCH_R):
            rb = bufs[rj % 2]
            row0 = rbase + rj * RC
            pltpu.make_async_copy(acc.at[pl.ds(row0, RC)], rb, semG0).wait()
            if rj + 1 < NCH_R:
                pltpu.async_copy(
                    acc.at[pl.ds(row0 + RC, RC)], bufs[(rj + 1) % 2], semG0)

            def _out_body(r, _, rj=rj, rb=rb):
                sv = dinv[rj * RC + r, :]
                for k in range(DH // 16):
                    rb[r, pl.ds(k * 16, 16)] = (
                        rb[r, pl.ds(k * 16, 16)] * sv + bvec[pl.ds(k * 16, 16)])
                return _
            lax.fori_loop(0, RC, _out_body, None)
            pltpu.async_copy(rb, outp.at[c, pl.ds(row0, RC)], semW)
            pltpu.make_async_copy(rb, outp.at[c, pl.ds(row0, RC)], semW).wait()
        plsc.subcore_barrier()


@jax.jit
def _sc_gcn(xw0, xw1, eir, bs):
    mesh = plsc.VectorSubcoreMesh(core_axis_name="c", subcore_axis_name="s")
    f = pl.kernel(
        _sc_gcn_body,
        out_type=[jax.ShapeDtypeStruct((2, NPAD, DH), jnp.float32),
                  jax.ShapeDtypeStruct((2, NPAD, DH), jnp.float32),
                  jax.ShapeDtypeStruct((2 * NPAD, DH), jnp.float32),
                  jax.ShapeDtypeStruct((2 * NPAD, DH), jnp.float32)],
        mesh=mesh,
        compiler_params=pltpu.CompilerParams(use_tc_tiling_on_sc=False),
        scratch_types=[
            pltpu.VMEM_SHARED((NPAD, DH), jnp.float32),
            pltpu.VMEM_SHARED((NPAD, 16), jnp.float32),
            pltpu.VMEM((8, EC), jnp.int32),
            pltpu.VMEM((8, EC), jnp.int32),
            pltpu.VMEM((EC, DH), jnp.float32),
            pltpu.VMEM((EC, DH), jnp.float32),
            pltpu.VMEM((EC, DH), jnp.float32),
            pltpu.VMEM((EC, DH), jnp.float32),
            pltpu.VMEM((EC, 16), jnp.float32),
            pltpu.VMEM((RT, 16), jnp.float32),
            pltpu.VMEM((RC, DH), jnp.float32),
            pltpu.VMEM((RC, DH), jnp.float32),
            pltpu.VMEM((DH,), jnp.float32),
            pltpu.SemaphoreType.DMA,
            pltpu.SemaphoreType.DMA,
            pltpu.SemaphoreType.DMA,
            pltpu.SemaphoreType.DMA,
            pltpu.SemaphoreType.DMA,
            pltpu.SemaphoreType.DMA,
            pltpu.SemaphoreType.DMA,
            pltpu.SemaphoreType.DMA,
            pltpu.SemaphoreType.DMA,
            pltpu.SemaphoreType.DMA,
            pltpu.SemaphoreType.DMA,
        ],
    )
    return f(xw0, xw1, eir, bs)


def _mm_body(x_ref, w_ref, o0_ref, o1_ref):
    r = jnp.dot(x_ref[0], w_ref[0], preferred_element_type=jnp.float32)
    o0_ref[0] = r[:, :DH]
    o1_ref[0] = r[:, DH:]


@jax.jit
def _tc_matmul(xs, Ws):
    BM = 1024
    return pl.pallas_call(
        _mm_body,
        grid=(2, NPAD // BM),
        in_specs=[pl.BlockSpec((1, BM, D), lambda b, i: (b, i, 0)),
                  pl.BlockSpec((1, D, D), lambda b, i: (b, 0, 0))],
        out_specs=[pl.BlockSpec((1, BM, DH), lambda b, i: (b, i, 0)),
                   pl.BlockSpec((1, BM, DH), lambda b, i: (b, i, 0))],
        out_shape=[jax.ShapeDtypeStruct((2, NPAD, DH), jnp.float32),
                   jax.ShapeDtypeStruct((2, NPAD, DH), jnp.float32)],
    )(xs, Ws)


def _pool_body(x_ref, o_ref):
    o_ref[...] = jnp.mean(x_ref[...], axis=1)


@jax.jit
def _tc_pool(cw):
    return pl.pallas_call(
        _pool_body,
        out_shape=jax.ShapeDtypeStruct((B, D), jnp.float32),
    )(cw)


def kernel(text_x, text_edge_index, text_batch, scene_x, scene_edge_index,
           scene_batch, W_text, b_text, W_scene, b_scene):
    xs = jnp.zeros((2, NPAD, D), jnp.float32).at[:, :N, :].set(
        jnp.stack([text_x, scene_x]))
    Ws = jnp.stack([W_text, W_scene])
    bs = jnp.stack([b_text, b_scene])
    ei = jnp.stack([text_edge_index, scene_edge_index])
    ei_p = jnp.concatenate(
        [ei, jnp.full((2, 2, EPAD - E), NPAD - 1, ei.dtype)], axis=-1)
    eir = ei_p.reshape(2, 2, 16, NCH_E, EC)

    xw0, xw1 = _tc_matmul(xs, Ws)
    out0, out1, _, _ = _sc_gcn(xw0, xw1, eir, bs)
    out = jnp.concatenate([out0, out1], axis=-1)

    contextual_words = out[0, :N].reshape(B, N // B, D)
    scene_graph_feats = out[1, :N].reshape(B, N // B, D)
    question = _tc_pool(contextual_words)
    return contextual_words, question, scene_graph_feats


# speedup vs baseline: 28.9036x; 1.0526x over previous
import jax
import jax.numpy as jnp
from jax import lax
from jax.experimental import pallas as pl
from jax.experimental.pallas import tpu as pltpu
from jax.experimental.pallas import tpu_sc as plsc

N = 10000
E = 320000
D = 128
B = 100

DH = D // 2
NPAD = 10240
RT = NPAD // 16
RC = 128
EC = 128
EPAD = 321536
ETP = EPAD // 16
NCH_E = ETP // EC
NCH_R = RT // RC


def _fast_rsqrt(d):
    xi = lax.bitcast_convert_type(d, jnp.int32)
    xi = jnp.int32(0x5F3759DF) - (xi >> 1)
    r = lax.bitcast_convert_type(xi, jnp.float32)
    r = r * (1.5 - 0.5 * d * r * r)
    r = r * (1.5 - 0.5 * d * r * r)
    r = r * (1.5 - 0.5 * d * r * r)
    return r


def _sc_gcn_body(xw0, xw1, eir, bs, outT, outS, y0, y1, acc, degsp,
                 sidx, didx, rows0, rows1, rows2, rows3, ones, dinv,
                 rbufA, rbufB, bvec,
                 semG0, semG1, semG2, semG3, semS0, semS1, semS2, semS3,
                 semD, semW, semI):
    c = lax.axis_index("c")
    s = lax.axis_index("s")
    boff = c * NPAD
    rows = (rows0, rows1, rows2, rows3)
    semG = (semG0, semG1, semG2, semG3)
    semS = (semS0, semS1, semS2, semS3)

    def _zero_body(r, _):
        dinv[r, :] = jnp.zeros((16,), jnp.float32)
        return _
    lax.fori_loop(0, RT, _zero_body, None)

    def _ones_body(r, _):
        ones[r, :] = jnp.ones((16,), jnp.float32)
        return _
    lax.fori_loop(0, EC, _ones_body, None)

    pltpu.sync_copy(dinv, degsp.at[pl.ds(RT * s, RT)])
    plsc.subcore_barrier()

    def _didx_load(j, sem):
        pltpu.async_copy(eir.at[c, 1, s, j], didx.at[j % 8], sem)

    def _didx_wait(j, sem):
        pltpu.make_async_copy(eir.at[c, 1, s, j], didx.at[j % 8], sem).wait()

    for j in range(4):
        _didx_load(j, semI)
    for j in range(4):
        _didx_wait(j, semI)
    for j in range(4, 8):
        _didx_load(j, semI)

    def _deg_body(q, _):
        j0 = 4 * q
        for b in range(4):
            pltpu.async_copy(ones, degsp.at[didx.at[(j0 + b) % 8]], semD,
                             add=True)
        for b in range(4):
            @pl.when(j0 + 4 + b < NCH_E)
            def _(b=b):
                _didx_wait(j0 + 4 + b, semI)
        for b in range(4):
            pltpu.make_async_copy(ones, degsp.at[didx.at[(j0 + b) % 8]],
                                  semD).wait()
        for b in range(4):
            @pl.when(j0 + 8 + b < NCH_E)
            def _(b=b):
                _didx_load(j0 + 8 + b, semI)
        return _
    lax.fori_loop(0, (NCH_E - 1) // 4, _deg_body, None)
    jl = NCH_E - 1
    pltpu.sync_copy(ones, degsp.at[didx.at[jl % 8]], add=True)
    plsc.subcore_barrier()

    pltpu.sync_copy(degsp.at[pl.ds(RT * s, RT)], dinv)

    def _rs_body(r, _):
        d = dinv[r, :] + 1.0
        dinv[r, :] = _fast_rsqrt(d)
        return _
    lax.fori_loop(0, RT, _rs_body, None)

    rbase = s * RT
    bufs = (rbufA, rbufB)
    for p, xwp, yp in ((0, xw0, y0), (1, xw1, y1)):
        pltpu.sync_copy(bs.at[c, pl.ds(p * DH, DH)], bvec)

        pltpu.async_copy(xwp.at[c, pl.ds(rbase, RC)], rbufA, semG0)
        for rj in range(NCH_R):
            rb = bufs[rj % 2]
            row0 = rbase + rj * RC
            pltpu.make_async_copy(xwp.at[c, pl.ds(row0, RC)], rb, semG0).wait()
            if rj + 1 < NCH_R:
                pltpu.async_copy(
                    xwp.at[c, pl.ds(row0 + RC, RC)], bufs[(rj + 1) % 2], semG0)

            def _scale_body(r, _, rj=rj, rb=rb):
                sv = dinv[rj * RC + r, :]
                for k in range(DH // 16):
                    rb[r, pl.ds(k * 16, 16)] = rb[r, pl.ds(k * 16, 16)] * sv
                return _
            lax.fori_loop(0, RC, _scale_body, None)
            pltpu.async_copy(rb, yp.at[pl.ds(c * NPAD + row0, RC)], semW)
            pltpu.sync_copy(rb, acc.at[pl.ds(row0, RC)])
            pltpu.make_async_copy(rb, yp.at[pl.ds(c * NPAD + row0, RC)], semW).wait()
        plsc.subcore_barrier()

        def _sidx_load(j, sem, yp=yp):
            pltpu.async_copy(eir.at[c, 0, s, j], sidx.at[j % 8], sem)

        def _idx_wait(j, sem, yp=yp):
            pltpu.make_async_copy(eir.at[c, 0, s, j], sidx.at[j % 8], sem).wait()
            pltpu.make_async_copy(eir.at[c, 1, s, j], didx.at[j % 8], sem).wait()

        def _gather(j, rb, sem, yp=yp):
            for k in range(EC // 16):
                sidx[j % 8, pl.ds(k * 16, 16)] = (
                    sidx[j % 8, pl.ds(k * 16, 16)] + boff)
            pltpu.async_copy(yp.at[sidx.at[j % 8]], rb, sem)

        def _gather_wait(j, rb, sem, yp=yp):
            pltpu.make_async_copy(yp.at[sidx.at[j % 8]], rb, sem).wait()

        def _scat(j, rb, sem):
            pltpu.async_copy(rb, acc.at[didx.at[j % 8]], sem, add=True)

        def _scat_wait(j, rb, sem):
            pltpu.make_async_copy(rb, acc.at[didx.at[j % 8]], sem).wait()

        for j in range(4):
            _sidx_load(j, semI)
            _didx_load(j, semI)
        for j in range(4):
            _idx_wait(j, semI)
        for b in range(4):
            _gather(b, rows[b], semG[b])
        for j in range(4, 8):
            _sidx_load(j, semI)
            _didx_load(j, semI)

        def _edge_body(q, _, yp=yp):
            j0 = 4 * q
            for b in range(4):
                _gather_wait(j0 + b, rows[b], semG[b])
                _scat(j0 + b, rows[b], semS[b])
            for b in range(4):
                @pl.when(j0 + 4 + b < NCH_E)
                def _(b=b):
                    _idx_wait(j0 + 4 + b, semI)
            for b in range(4):
                _scat_wait(j0 + b, rows[b], semS[b])

                @pl.when(j0 + 4 + b < NCH_E)
                def _(b=b):
                    _gather(j0 + 4 + b, rows[b], semG[b])
            for b in range(4):
                @pl.when(j0 + 8 + b < NCH_E)
                def _(b=b):
                    _sidx_load(j0 + 8 + b, semI)
                    _didx_load(j0 + 8 + b, semI)
            return _
        lax.fori_loop(0, (NCH_E - 1) // 4, _edge_body, None)
        jl = NCH_E - 1
        _gather_wait(jl, rows[0], semG[0])
        pltpu.sync_copy(rows[0], acc.at[didx.at[jl % 8]], add=True)
        plsc.subcore_barrier()

        pltpu.async_copy(acc.at[pl.ds(rbase, RC)], rbufA, semG0)
        for rj in range(NCH_R):
            rb = bufs[rj % 2]
            row0 = rbase + rj * RC
            pltpu.make_async_copy(acc.at[pl.ds(row0, RC)], rb, semG0).wait()
            if rj + 1 < NCH_R:
                pltpu.async_copy(
                    acc.at[pl.ds(row0 + RC, RC)], bufs[(rj + 1) % 2], semG0)

            def _out_body(r, _, rj=rj, rb=rb):
                sv = dinv[rj * RC + r, :]
                for k in range(DH // 16):
                    rb[r, pl.ds(k * 16, 16)] = (
                        rb[r, pl.ds(k * 16, 16)] * sv + bvec[pl.ds(k * 16, 16)])
                return _
            lax.fori_loop(0, RC, _out_body, None)
            for cc, outp in ((0, outT), (1, outS)):
                @pl.when((c == cc) & (row0 + RC <= N))
                def _(outp=outp, row0=row0, rb=rb, p=p):
                    pltpu.sync_copy(
                        rb, outp.at[pl.ds(row0, RC), pl.ds(p * DH, DH)])

                @pl.when((c == cc) & (row0 < N) & (row0 + RC > N))
                def _(outp=outp, row0=row0, rb=rb, p=p):
                    pltpu.sync_copy(
                        rb.at[pl.ds(0, N % RC)],
                        outp.at[pl.ds(row0, N % RC), pl.ds(p * DH, DH)])
        plsc.subcore_barrier()


@jax.jit
def _sc_gcn(xw0, xw1, eir, bs):
    mesh = plsc.VectorSubcoreMesh(core_axis_name="c", subcore_axis_name="s")
    f = pl.kernel(
        _sc_gcn_body,
        out_type=[jax.ShapeDtypeStruct((N, D), jnp.float32),
                  jax.ShapeDtypeStruct((N, D), jnp.float32),
                  jax.ShapeDtypeStruct((2 * NPAD, DH), jnp.float32),
                  jax.ShapeDtypeStruct((2 * NPAD, DH), jnp.float32)],
        mesh=mesh,
        compiler_params=pltpu.CompilerParams(use_tc_tiling_on_sc=False),
        scratch_types=[
            pltpu.VMEM_SHARED((NPAD, DH), jnp.float32),
            pltpu.VMEM_SHARED((NPAD, 16), jnp.float32),
            pltpu.VMEM((8, EC), jnp.int32),
            pltpu.VMEM((8, EC), jnp.int32),
            pltpu.VMEM((EC, DH), jnp.float32),
            pltpu.VMEM((EC, DH), jnp.float32),
            pltpu.VMEM((EC, DH), jnp.float32),
            pltpu.VMEM((EC, DH), jnp.float32),
            pltpu.VMEM((EC, 16), jnp.float32),
            pltpu.VMEM((RT, 16), jnp.float32),
            pltpu.VMEM((RC, DH), jnp.float32),
            pltpu.VMEM((RC, DH), jnp.float32),
            pltpu.VMEM((DH,), jnp.float32),
            pltpu.SemaphoreType.DMA,
            pltpu.SemaphoreType.DMA,
            pltpu.SemaphoreType.DMA,
            pltpu.SemaphoreType.DMA,
            pltpu.SemaphoreType.DMA,
            pltpu.SemaphoreType.DMA,
            pltpu.SemaphoreType.DMA,
            pltpu.SemaphoreType.DMA,
            pltpu.SemaphoreType.DMA,
            pltpu.SemaphoreType.DMA,
            pltpu.SemaphoreType.DMA,
        ],
    )
    return f(xw0, xw1, eir, bs)


def _mm_body(x_ref, w_ref, o0_ref, o1_ref):
    r = jnp.dot(x_ref[0], w_ref[0], preferred_element_type=jnp.float32)
    o0_ref[0] = r[:, :DH]
    o1_ref[0] = r[:, DH:]


@jax.jit
def _tc_matmul(xs, Ws):
    BM = 1280
    return pl.pallas_call(
        _mm_body,
        grid=(2, NPAD // BM),
        in_specs=[pl.BlockSpec((1, BM, D), lambda b, i: (b, i, 0)),
                  pl.BlockSpec((1, D, D), lambda b, i: (b, 0, 0))],
        out_specs=[pl.BlockSpec((1, BM, DH), lambda b, i: (b, i, 0)),
                   pl.BlockSpec((1, BM, DH), lambda b, i: (b, i, 0))],
        out_shape=[jax.ShapeDtypeStruct((2, NPAD, DH), jnp.float32),
                   jax.ShapeDtypeStruct((2, NPAD, DH), jnp.float32)],
    )(xs, Ws)


def _pool_body(x_ref, o_ref):
    o_ref[...] = jnp.mean(x_ref[...], axis=1)


@jax.jit
def _tc_pool(cw):
    return pl.pallas_call(
        _pool_body,
        out_shape=jax.ShapeDtypeStruct((B, D), jnp.float32),
    )(cw)


def kernel(text_x, text_edge_index, text_batch, scene_x, scene_edge_index,
           scene_batch, W_text, b_text, W_scene, b_scene):
    xs = jnp.stack([text_x, scene_x])
    Ws = jnp.stack([W_text, W_scene])
    bs = jnp.stack([b_text, b_scene])
    ei = jnp.stack([text_edge_index, scene_edge_index])
    ei_p = jnp.concatenate(
        [ei, jnp.full((2, 2, EPAD - E), NPAD - 1, ei.dtype)], axis=-1)
    eir = ei_p.reshape(2, 2, 16, NCH_E, EC)

    xw0, xw1 = _tc_matmul(xs, Ws)
    out_t, out_s, _, _ = _sc_gcn(xw0, xw1, eir, bs)

    contextual_words = out_t.reshape(B, N // B, D)
    scene_graph_feats = out_s.reshape(B, N // B, D)
    question = _tc_pool(contextual_words)
    return contextual_words, question, scene_graph_feats
